# baseline (speedup 1.0000x reference)
import functools

import jax
import jax.numpy as jnp
from jax import lax
from jax.experimental import pallas as pl
from jax.experimental.pallas import tpu as pltpu
from jax.experimental.pallas import tpu_sc as plsc

N = 10000
E = 320000
D_FEAT = 128
H1 = 16
H2 = 32

NC = 2
NS = 16
L = 16
NW = NC * NS

BLK = 128
BLKS_PER_TILE = 80
EPAD = NW * BLKS_PER_TILE * BLK
NPAD = 10112
RPT = NPAD // NS

@functools.cache
def _mesh():
    return plsc.VectorSubcoreMesh(core_axis_name="c", subcore_axis_name="s",
                                  num_cores=NC, num_subcores=NS)


def _fill_zeros(ref, nrows, ncols):
    @pl.loop(0, nrows)
    def _z(r):
        for hh in range(ncols // L):
            ref[r, pl.ds(hh * L, L)] = jnp.zeros((L,), jnp.float32)


NBUF = 4


@functools.cache
def _make_agg(H):
    @functools.partial(
        pl.kernel,
        out_type=jax.ShapeDtypeStruct((NC, NPAD, H), jnp.float32),
        mesh=_mesh(),
        compiler_params=pltpu.CompilerParams(use_tc_tiling_on_sc=False),
        scratch_types=[
            pltpu.VMEM((BLKS_PER_TILE, BLK), jnp.int32),
            pltpu.VMEM((BLKS_PER_TILE, BLK), jnp.int32),
            pltpu.VMEM((RPT, H), jnp.float32),
            pltpu.VMEM_SHARED((NPAD, H), jnp.float32),
        ]
        + [pltpu.VMEM((BLK, H), jnp.float32) for _ in range(NBUF)]
        + [pltpu.SemaphoreType.DMA for _ in range(NBUF)],
    )
    def agg(hs, srcp, dstp, out, src_v, dst_v, zer_v, acc_sh, *rb):
        rows = rb[:NBUF]
        sems = rb[NBUF:]
        cid = lax.axis_index("c")
        sid = lax.axis_index("s")
        w = cid * NS + sid

        _fill_zeros(zer_v, RPT, H)
        pltpu.sync_copy(zer_v, acc_sh.at[pl.ds(sid * RPT, RPT)])
        plsc.subcore_barrier()

        base = w * BLKS_PER_TILE
        pltpu.sync_copy(srcp.at[pl.ds(base, BLKS_PER_TILE)], src_v)
        pltpu.sync_copy(dstp.at[pl.ds(base, BLKS_PER_TILE)], dst_v)

        for b in range(NBUF):
            pltpu.async_copy(hs.at[src_v.at[b]], rows[b], sems[b])

        @pl.loop(0, BLKS_PER_TILE, step=NBUF)
        def _chunk(j):
            for b in range(NBUF):
                pltpu.make_async_copy(hs.at[pl.ds(0, BLK)], rows[b],
                                      sems[b]).wait()
                pltpu.sync_copy(rows[b], acc_sh.at[dst_v.at[j + b]], add=True)

                @pl.when(j + b + NBUF < BLKS_PER_TILE)
                def _refill(b=b):
                    pltpu.async_copy(hs.at[src_v.at[j + b + NBUF]],
                                     rows[b], sems[b])

        plsc.subcore_barrier()
        pltpu.sync_copy(acc_sh.at[pl.ds(sid * RPT, RPT)],
                        out.at[cid, pl.ds(sid * RPT, RPT)])

    return agg


DEGL = 8


@functools.cache
def _make_deg():
    @functools.partial(
        pl.kernel,
        out_type=jax.ShapeDtypeStruct((NC, NPAD, DEGL), jnp.float32),
        mesh=_mesh(),
        compiler_params=pltpu.CompilerParams(use_tc_tiling_on_sc=False),
        scratch_types=[
            pltpu.VMEM((BLKS_PER_TILE, BLK), jnp.int32),
            pltpu.VMEM((BLK, DEGL), jnp.float32),
            pltpu.VMEM_SHARED((NPAD, DEGL), jnp.float32),
        ],
    )
    def _deg(dstp, ones8, zer8, out, dst_v, ones_v, acc_sh):
        cid = lax.axis_index("c")
        sid = lax.axis_index("s")
        w = cid * NS + sid

        pltpu.sync_copy(ones8, ones_v)
        pltpu.sync_copy(zer8, acc_sh.at[pl.ds(sid * RPT, RPT)])
        plsc.subcore_barrier()

        pltpu.sync_copy(dstp.at[pl.ds(w * BLKS_PER_TILE, BLKS_PER_TILE)], dst_v)

        @pl.loop(0, BLKS_PER_TILE)
        def _edge_block(j):
            pltpu.sync_copy(ones_v, acc_sh.at[dst_v.at[j]], add=True)

        plsc.subcore_barrier()
        pltpu.sync_copy(acc_sh.at[pl.ds(sid * RPT, RPT)],
                        out.at[cid, pl.ds(sid * RPT, RPT)])

    return _deg


def _tc0_body(x_ref, w1_ref, h1_ref):
    h1_ref[...] = jnp.dot(x_ref[...], w1_ref[...],
                          preferred_element_type=jnp.float32)


def _tc1_body(degp_ref, h1_ref, h1s_ref, ds_ref):
    deg = jnp.mean(degp_ref[0, :N, :] + degp_ref[1, :N, :],
                   axis=-1, keepdims=True) + 1.0
    ds = lax.rsqrt(deg)
    h1s_ref[...] = h1_ref[...] * ds
    ds_ref[...] = ds


def _tc2_body(p_ref, h1s_ref, ds_ref, b1_ref, w2_ref, h2s_ref):
    agg = p_ref[0, :N, :] + p_ref[1, :N, :] + h1s_ref[...]
    z1 = jnp.maximum(agg * ds_ref[...] + b1_ref[...], 0.0)
    h2 = jnp.dot(z1, w2_ref[...], preferred_element_type=jnp.float32)
    h2s_ref[...] = h2 * ds_ref[...]


def _tc3_body(q_ref, h2s_ref, ds_ref, b2_ref, wfc_ref, bfc_ref, out_ref):
    agg = q_ref[0, :N, :] + q_ref[1, :N, :] + h2s_ref[...]
    z2 = jnp.maximum(agg * ds_ref[...] + b2_ref[...], 0.0)
    g = jnp.mean(z2, axis=0)
    out_ref[...] = jnp.sum(g[:, None] * wfc_ref[...]) + bfc_ref[...]


def kernel(x, edge_index, W1, b1, W2, b2, Wfc, bfc):
    ei = edge_index.astype(jnp.int32)
    pad = EPAD - E
    ar = jnp.arange(pad, dtype=jnp.int32)
    srcp = jnp.concatenate([ei[0], (ar * 97) % N])
    dstp = jnp.concatenate([ei[1], N + ar % (NPAD - N)])
    srcp = srcp.reshape(EPAD // BLK, BLK)
    dstp = dstp.reshape(EPAD // BLK, BLK)

    ones8 = jnp.ones((BLK, DEGL), jnp.float32)
    zer8 = jnp.zeros((RPT, DEGL), jnp.float32)
    degp = _make_deg()(dstp, ones8, zer8)

    h1 = pl.pallas_call(
        _tc0_body,
        out_shape=jax.ShapeDtypeStruct((N, H1), jnp.float32),
    )(x, W1)

    h1s, ds = pl.pallas_call(
        _tc1_body,
        out_shape=[
            jax.ShapeDtypeStruct((N, H1), jnp.float32),
            jax.ShapeDtypeStruct((N, 1), jnp.float32),
        ],
    )(degp, h1)

    p1 = _make_agg(H1)(h1s, srcp, dstp)

    h2s = pl.pallas_call(
        _tc2_body,
        out_shape=jax.ShapeDtypeStruct((N, H2), jnp.float32),
    )(p1, h1s, ds, b1, W2)

    p2 = _make_agg(H2)(h2s, srcp, dstp)

    out = pl.pallas_call(
        _tc3_body,
        out_shape=jax.ShapeDtypeStruct((1,), jnp.float32),
    )(p2, h2s, ds, b2, Wfc, bfc)

    return out


# speedup vs baseline: 1.0570x; 1.0570x over previous
import functools

import jax
import jax.numpy as jnp
from jax import lax
from jax.experimental import pallas as pl
from jax.experimental.pallas import tpu as pltpu
from jax.experimental.pallas import tpu_sc as plsc

N = 10000
E = 320000
D_FEAT = 128
H1 = 16
H2 = 32

NC = 2
NS = 16
L = 16
NW = NC * NS

BLK = 128
BLKS_PER_TILE = 80
EPAD = NW * BLKS_PER_TILE * BLK
NPAD = 10112
RPT = NPAD // NS

@functools.cache
def _mesh():
    return plsc.VectorSubcoreMesh(core_axis_name="c", subcore_axis_name="s",
                                  num_cores=NC, num_subcores=NS)


def _fill_zeros(ref, nrows, ncols):
    @pl.loop(0, nrows)
    def _z(r):
        for hh in range(ncols // L):
            ref[r, pl.ds(hh * L, L)] = jnp.zeros((L,), jnp.float32)


NBUF = 8


@functools.cache
def _make_agg(H):
    @functools.partial(
        pl.kernel,
        out_type=jax.ShapeDtypeStruct((NC, NPAD, H), jnp.float32),
        mesh=_mesh(),
        compiler_params=pltpu.CompilerParams(use_tc_tiling_on_sc=False),
        scratch_types=[
            pltpu.VMEM((BLKS_PER_TILE, BLK), jnp.int32),
            pltpu.VMEM((BLKS_PER_TILE, BLK), jnp.int32),
            pltpu.VMEM((RPT, H), jnp.float32),
            pltpu.VMEM_SHARED((NPAD, H), jnp.float32),
        ]
        + [pltpu.VMEM((BLK, H), jnp.float32) for _ in range(NBUF)]
        + [pltpu.SemaphoreType.DMA for _ in range(2 * NBUF)],
    )
    def agg(hs, srcp, dstp, out, src_v, dst_v, zer_v, acc_sh, *rb):
        rows = rb[:NBUF]
        gsem = rb[NBUF:2 * NBUF]
        ssem = rb[2 * NBUF:]
        cid = lax.axis_index("c")
        sid = lax.axis_index("s")
        w = cid * NS + sid

        _fill_zeros(zer_v, RPT, H)
        pltpu.sync_copy(zer_v, acc_sh.at[pl.ds(sid * RPT, RPT)])
        plsc.subcore_barrier()

        base = w * BLKS_PER_TILE
        pltpu.sync_copy(srcp.at[pl.ds(base, BLKS_PER_TILE)], src_v)
        pltpu.sync_copy(dstp.at[pl.ds(base, BLKS_PER_TILE)], dst_v)

        for b in range(NBUF):
            pltpu.async_copy(hs.at[src_v.at[b]], rows[b], gsem[b])

        @pl.loop(0, BLKS_PER_TILE, step=NBUF)
        def _chunk(j):
            for b in range(NBUF):
                pltpu.make_async_copy(hs.at[pl.ds(0, BLK)], rows[b],
                                      gsem[b]).wait()
                pltpu.async_copy(rows[b], acc_sh.at[dst_v.at[j + b]],
                                 ssem[b], add=True)
            for b in range(NBUF):
                pltpu.make_async_copy(rows[b], acc_sh.at[dst_v.at[j]],
                                      ssem[b]).wait()

                @pl.when(j + b + NBUF < BLKS_PER_TILE)
                def _refill(b=b):
                    pltpu.async_copy(hs.at[src_v.at[j + b + NBUF]],
                                     rows[b], gsem[b])

        plsc.subcore_barrier()
        pltpu.sync_copy(acc_sh.at[pl.ds(sid * RPT, RPT)],
                        out.at[cid, pl.ds(sid * RPT, RPT)])

    return agg


DEGL = 8


@functools.cache
def _make_deg():
    @functools.partial(
        pl.kernel,
        out_type=jax.ShapeDtypeStruct((NC, NPAD, DEGL), jnp.float32),
        mesh=_mesh(),
        compiler_params=pltpu.CompilerParams(use_tc_tiling_on_sc=False),
        scratch_types=[
            pltpu.VMEM((BLKS_PER_TILE, BLK), jnp.int32),
            pltpu.VMEM((BLK, DEGL), jnp.float32),
            pltpu.VMEM_SHARED((NPAD, DEGL), jnp.float32),
            pltpu.SemaphoreType.DMA,
        ],
    )
    def _deg(dstp, ones8, zer8, out, dst_v, ones_v, acc_sh, dsem):
        cid = lax.axis_index("c")
        sid = lax.axis_index("s")
        w = cid * NS + sid

        pltpu.sync_copy(ones8, ones_v)
        pltpu.sync_copy(zer8, acc_sh.at[pl.ds(sid * RPT, RPT)])
        plsc.subcore_barrier()

        pltpu.sync_copy(dstp.at[pl.ds(w * BLKS_PER_TILE, BLKS_PER_TILE)], dst_v)

        @pl.loop(0, BLKS_PER_TILE)
        def _edge_block(j):
            pltpu.async_copy(ones_v, acc_sh.at[dst_v.at[j]], dsem, add=True)

        @pl.loop(0, BLKS_PER_TILE)
        def _drain(j):
            pltpu.make_async_copy(ones_v, acc_sh.at[dst_v.at[0]], dsem).wait()

        plsc.subcore_barrier()
        pltpu.sync_copy(acc_sh.at[pl.ds(sid * RPT, RPT)],
                        out.at[cid, pl.ds(sid * RPT, RPT)])

    return _deg


def _tc0_body(x_ref, w1_ref, h1_ref):
    h1_ref[...] = jnp.dot(x_ref[...], w1_ref[...],
                          preferred_element_type=jnp.float32)


def _tc1_body(degp_ref, h1_ref, h1s_ref, ds_ref):
    deg = jnp.mean(degp_ref[0, :N, :] + degp_ref[1, :N, :],
                   axis=-1, keepdims=True) + 1.0
    ds = lax.rsqrt(deg)
    h1s_ref[...] = h1_ref[...] * ds
    ds_ref[...] = ds


def _tc2_body(p_ref, h1s_ref, ds_ref, b1_ref, w2_ref, h2s_ref):
    agg = p_ref[0, :N, :] + p_ref[1, :N, :] + h1s_ref[...]
    z1 = jnp.maximum(agg * ds_ref[...] + b1_ref[...], 0.0)
    h2 = jnp.dot(z1, w2_ref[...], preferred_element_type=jnp.float32)
    h2s_ref[...] = h2 * ds_ref[...]


def _tc3_body(q_ref, h2s_ref, ds_ref, b2_ref, wfc_ref, bfc_ref, out_ref):
    agg = q_ref[0, :N, :] + q_ref[1, :N, :] + h2s_ref[...]
    z2 = jnp.maximum(agg * ds_ref[...] + b2_ref[...], 0.0)
    g = jnp.mean(z2, axis=0)
    out_ref[...] = jnp.sum(g[:, None] * wfc_ref[...]) + bfc_ref[...]


def kernel(x, edge_index, W1, b1, W2, b2, Wfc, bfc):
    ei = edge_index.astype(jnp.int32)
    pad = EPAD - E
    ar = jnp.arange(pad, dtype=jnp.int32)
    srcp = jnp.concatenate([ei[0], (ar * 97) % N])
    dstp = jnp.concatenate([ei[1], N + ar % (NPAD - N)])
    srcp = srcp.reshape(EPAD // BLK, BLK)
    dstp = dstp.reshape(EPAD // BLK, BLK)

    ones8 = jnp.ones((BLK, DEGL), jnp.float32)
    zer8 = jnp.zeros((RPT, DEGL), jnp.float32)
    degp = _make_deg()(dstp, ones8, zer8)

    h1 = pl.pallas_call(
        _tc0_body,
        out_shape=jax.ShapeDtypeStruct((N, H1), jnp.float32),
    )(x, W1)

    h1s, ds = pl.pallas_call(
        _tc1_body,
        out_shape=[
            jax.ShapeDtypeStruct((N, H1), jnp.float32),
            jax.ShapeDtypeStruct((N, 1), jnp.float32),
        ],
    )(degp, h1)

    p1 = _make_agg(H1)(h1s, srcp, dstp)

    h2s = pl.pallas_call(
        _tc2_body,
        out_shape=jax.ShapeDtypeStruct((N, H2), jnp.float32),
    )(p1, h1s, ds, b1, W2)

    p2 = _make_agg(H2)(h2s, srcp, dstp)

    out = pl.pallas_call(
        _tc3_body,
        out_shape=jax.ShapeDtypeStruct((1,), jnp.float32),
    )(p2, h2s, ds, b2, Wfc, bfc)

    return out
